# baseline (speedup 1.0000x reference)
import functools

import jax
import jax.numpy as jnp
from jax import lax
from jax.experimental import pallas as pl
from jax.experimental.pallas import tpu as pltpu
from jax.experimental.pallas import tpu_sc as plsc

N = 10000
E = 320000
H = 128
B = 50
K_POOL = 30
KS = 3
L1 = K_POOL - KS + 1
C1 = 32
DL1 = 32
DL2 = 64
NUM_OUT = 10

NP = 10240
NC, NS = 2, 16
NW = NC * NS
CH = 120
NCH = 84
NPAIR = NCH // 2
EPW = NCH * CH
EP = NW * EPW
NACC = 10112
RTA = NACC // NS
RB = 512
GR = 4608
GPW = GR // NW
_CC = (120, 120, 120, 120, 120, 32)


def _sc_mesh():
    return plsc.VectorSubcoreMesh(
        core_axis_name="c", subcore_axis_name="s", num_cores=NC, num_subcores=NS)


def _fill_tile(buf, w, val):
    v16 = jnp.full((16,), val, jnp.float32)

    def frow(r, _):
        for v in range(w // 16):
            buf[r, pl.ds(v * 16, 16)] = v16
        return _
    lax.fori_loop(0, CH, frow, 0)


def _zero_acc(rows0, w, acc, s):
    _fill_tile(rows0, w, 0.0)
    off = 0
    for sz in _CC:
        pltpu.sync_copy(rows0.at[pl.ds(0, sz)],
                        acc.at[pl.ds(s * RTA + off, sz)])
        off += sz


def _copy_out(acc, out_hbm, c, s, so):
    off = 0
    for sz in _CC:
        r0 = s * RTA + off
        pltpu.async_copy(acc.at[pl.ds(r0, sz)], out_hbm.at[c, pl.ds(r0, sz)], so)
        off += sz
    off = 0
    for sz in _CC:
        pltpu.make_async_copy(acc.at[pl.ds(0, sz)],
                              out_hbm.at[0, pl.ds(0, sz)], so).wait()
        off += sz


GS = (64, 56)


def _make_seg_sum():
    def body(h_hbm, src_hbm, dst_hbm, parts_hbm,
             sb0a, sb0b, sb1a, sb1b, db0a, db0b, db1a, db1b,
             rows0, rows1, acc, si0, si1, sg0, sg1, ss0, ss1, so):
        c = lax.axis_index("c")
        s = lax.axis_index("s")
        wid = s * NC + c
        ebase = wid * EPW
        sbufs = ((sb0a, sb1a), (sb0b, sb1b))
        dbufs = ((db0a, db1a), (db0b, db1b))
        isems = (si0, si1)

        def idx_fetch(p, q):
            for k in range(2):
                base = ebase + (2 * p + k) * CH
                pltpu.async_copy(src_hbm.at[pl.ds(base, CH)], sbufs[q][k], isems[q])
                pltpu.async_copy(dst_hbm.at[pl.ds(base, CH)], dbufs[q][k], isems[q])

        def idx_wait(q):
            for _ in range(4):
                pltpu.make_async_copy(src_hbm.at[pl.ds(0, CH)], sbufs[q][0],
                                      isems[q]).wait()

        def gather(q, k, buf, sem):
            off = 0
            for sz in GS:
                pltpu.async_copy(h_hbm.at[sbufs[q][k].at[pl.ds(off, sz)]],
                                 buf.at[pl.ds(off, sz)], sem)
                off += sz

        def wait_g(buf, sem):
            for sz in GS:
                pltpu.make_async_copy(h_hbm.at[pl.ds(0, sz)],
                                      buf.at[pl.ds(0, sz)], sem).wait()

        def drain_s(buf, sem):
            pltpu.make_async_copy(buf, acc.at[pl.ds(0, CH)], sem).wait()

        idx_fetch(0, 0)
        _zero_acc(rows0, H, acc, s)
        plsc.subcore_barrier()

        def do_pair(p, q):
            idx_wait(q)

            @pl.when(p > 0)
            def _drains():
                drain_s(rows0, ss0)
                drain_s(rows1, ss1)
            gather(q, 0, rows0, sg0)
            gather(q, 1, rows1, sg1)

            @pl.when(p + 1 < NPAIR)
            def _prefetch():
                idx_fetch(p + 1, 1 - q)
            wait_g(rows0, sg0)
            pltpu.async_copy(rows0, acc.at[dbufs[q][0]], ss0, add=True)
            wait_g(rows1, sg1)
            pltpu.async_copy(rows1, acc.at[dbufs[q][1]], ss1, add=True)

        def super_it(u, _):
            do_pair(2 * u, 0)
            do_pair(2 * u + 1, 1)
            return _
        lax.fori_loop(0, NPAIR // 2, super_it, 0)
        drain_s(rows0, ss0)
        drain_s(rows1, ss1)
        plsc.subcore_barrier()

        _copy_out(acc, parts_hbm, c, s, so)

    return pl.kernel(
        body,
        out_type=jax.ShapeDtypeStruct((NC, NP, H), jnp.float32),
        mesh=_sc_mesh(),
        scratch_types=(
            *[pltpu.VMEM((CH,), jnp.int32) for _ in range(8)],
            pltpu.VMEM((CH, H), jnp.float32),
            pltpu.VMEM((CH, H), jnp.float32),
            pltpu.VMEM_SHARED((NACC, H), jnp.float32),
            pltpu.SemaphoreType.DMA,
            pltpu.SemaphoreType.DMA,
            pltpu.SemaphoreType.DMA,
            pltpu.SemaphoreType.DMA,
            pltpu.SemaphoreType.DMA,
            pltpu.SemaphoreType.DMA,
            pltpu.SemaphoreType.DMA,
        ))


def _make_deg():
    def body(dst_hbm, degp_hbm, db0a, db0b, db1a, db1b, ones_tile, dacc,
             si0, si1, ss0, ss1, so):
        c = lax.axis_index("c")
        s = lax.axis_index("s")
        wid = s * NC + c
        ebase = wid * EPW
        dbufs = ((db0a, db1a), (db0b, db1b))
        isems = (si0, si1)

        _zero_acc(ones_tile, H, dacc, s)
        _fill_tile(ones_tile, H, 1.0)
        plsc.subcore_barrier()

        def idx_fetch(p, q):
            for k in range(2):
                base = ebase + (2 * p + k) * CH
                pltpu.async_copy(dst_hbm.at[pl.ds(base, CH)], dbufs[q][k], isems[q])

        def idx_wait(q):
            for _ in range(2):
                pltpu.make_async_copy(dst_hbm.at[pl.ds(0, CH)], dbufs[q][0],
                                      isems[q]).wait()

        def drain_s(sem):
            pltpu.make_async_copy(ones_tile, dacc.at[pl.ds(0, CH)], sem).wait()

        idx_fetch(0, 0)

        def do_pair(p, q):
            idx_wait(q)

            @pl.when(p > 0)
            def _drains():
                drain_s(ss0)
                drain_s(ss1)

            @pl.when(p + 1 < NCH // 2)
            def _prefetch():
                idx_fetch(p + 1, 1 - q)
            pltpu.async_copy(ones_tile, dacc.at[dbufs[q][0]], ss0, add=True)
            pltpu.async_copy(ones_tile, dacc.at[dbufs[q][1]], ss1, add=True)

        def super_it(u, _):
            do_pair(2 * u, 0)
            do_pair(2 * u + 1, 1)
            return _
        lax.fori_loop(0, NCH // 4, super_it, 0)
        drain_s(ss0)
        drain_s(ss1)
        plsc.subcore_barrier()

        _copy_out(dacc, degp_hbm, c, s, so)

    return pl.kernel(
        body,
        out_type=jax.ShapeDtypeStruct((NC, NP, H), jnp.float32),
        mesh=_sc_mesh(),
        scratch_types=(
            pltpu.VMEM((CH,), jnp.int32),
            pltpu.VMEM((CH,), jnp.int32),
            pltpu.VMEM((CH,), jnp.int32),
            pltpu.VMEM((CH,), jnp.int32),
            pltpu.VMEM((CH, H), jnp.float32),
            pltpu.VMEM_SHARED((NACC, H), jnp.float32),
            pltpu.SemaphoreType.DMA,
            pltpu.SemaphoreType.DMA,
            pltpu.SemaphoreType.DMA,
            pltpu.SemaphoreType.DMA,
            pltpu.SemaphoreType.DMA,
        ))


def _gather_rows():
    def body(h_hbm, gidx_hbm, out_hbm, idx_a, idx_b, rows, sem):
        c = lax.axis_index("c")
        s = lax.axis_index("s")
        base = (s * NC + c) * GPW
        half = GPW // 2
        pltpu.sync_copy(gidx_hbm.at[pl.ds(base, half)], idx_a)
        pltpu.sync_copy(gidx_hbm.at[pl.ds(base + half, half)], idx_b)
        pltpu.async_copy(h_hbm.at[idx_a], rows, sem).wait()
        pltpu.sync_copy(rows, out_hbm.at[pl.ds(base, half)])
        pltpu.async_copy(h_hbm.at[idx_b], rows, sem).wait()
        pltpu.sync_copy(rows, out_hbm.at[pl.ds(base + half, half)])

    return pl.kernel(
        body,
        out_type=jax.ShapeDtypeStruct((GR, H), jnp.float32),
        mesh=_sc_mesh(),
        scratch_types=(
            pltpu.VMEM((GPW // 2,), jnp.int32),
            pltpu.VMEM((GPW // 2,), jnp.int32),
            pltpu.VMEM((GPW // 2, H), jnp.float32),
            pltpu.SemaphoreType.DMA,
        ))


def _hr_body(h_ref, wr_ref, b_ref, o_ref):
    o_ref[...] = h_ref[...] @ wr_ref[...] + b_ref[...]


_hr = pl.pallas_call(
    _hr_body,
    grid=(NP // RB,),
    in_specs=[
        pl.BlockSpec((RB, H), lambda i: (i, 0)),
        pl.BlockSpec((H, H), lambda i: (0, 0)),
        pl.BlockSpec((1, H), lambda i: (0, 0)),
    ],
    out_specs=pl.BlockSpec((RB, H), lambda i: (i, 0)),
    out_shape=jax.ShapeDtypeStruct((NP, H), jnp.float32),
)


def _layer_body(emit_key, hp_ref, hr_ref, dp_ref, wl_ref, *outs):
    i = pl.program_id(0)
    agg = hp_ref[0] + hp_ref[1]
    d = dp_ref[0, :, 0:1] + dp_ref[1, :, 0:1]
    mean = agg / jnp.maximum(d, 1.0)
    out = mean @ wl_ref[...] + hr_ref[...]
    out = jnp.maximum(out, 0.0)
    rid = i * RB + lax.broadcasted_iota(jnp.int32, (RB, 1), 0)
    out = jnp.where(rid < N, out, 0.0)
    outs[0][...] = out
    if emit_key:
        outs[1][...] = out[:, H - 1:H]


def _make_layer(emit_key):
    out_shape = [jax.ShapeDtypeStruct((NP, H), jnp.float32)]
    out_specs = [pl.BlockSpec((RB, H), lambda i: (i, 0))]
    if emit_key:
        out_shape.append(jax.ShapeDtypeStruct((NP, 1), jnp.float32))
        out_specs.append(pl.BlockSpec((RB, 1), lambda i: (i, 0)))
    return pl.pallas_call(
        functools.partial(_layer_body, emit_key),
        grid=(NP // RB,),
        in_specs=[
            pl.BlockSpec((NC, RB, H), lambda i: (0, i, 0)),
            pl.BlockSpec((RB, H), lambda i: (i, 0)),
            pl.BlockSpec((NC, RB, H), lambda i: (0, i, 0)),
            pl.BlockSpec((H, H), lambda i: (0, 0)),
        ],
        out_specs=out_specs if emit_key else out_specs[0],
        out_shape=out_shape if emit_key else out_shape[0],
    )


def _sortpool_body(key_ref, batch_ref, out_ref, keys_scr):
    giota = lax.broadcasted_iota(jnp.int32, (NP, 64), 1)
    riota = lax.broadcasted_iota(jnp.int32, (NP, 64), 0)
    mask = batch_ref[...] == giota
    keys_scr[...] = jnp.where(mask, key_ref[...], -1.0)

    def it(k, out):
        kv = keys_scr[...]
        m = jnp.max(kv, axis=0, keepdims=True)
        cand = jnp.where(kv == m, riota, NP)
        sel = jnp.min(cand, axis=0, keepdims=True)
        sel = jnp.where(m > -0.5, sel, N)
        keys_scr[...] = jnp.where(riota == sel, -1.0, kv)
        out = jnp.where(
            lax.broadcasted_iota(jnp.int32, (32, 64), 0) == k, sel, out)
        return out

    out_ref[...] = lax.fori_loop(0, K_POOL, it, jnp.zeros((32, 64), jnp.int32))


_sortpool = pl.pallas_call(
    _sortpool_body,
    out_shape=jax.ShapeDtypeStruct((32, 64), jnp.int32),
    scratch_shapes=[pltpu.VMEM((NP, 64), jnp.float32)],
)


def _conv_body(u_ref, w_ref, b_ref, o_ref):
    o_ref[...] = jnp.maximum(u_ref[...] @ w_ref[...] + b_ref[...], 0.0)


_conv = pl.pallas_call(
    _conv_body,
    out_shape=jax.ShapeDtypeStruct((1408, H), jnp.float32),
)


def _mlp_body(y_ref, w1_ref, b1_ref, w2_ref, b2_ref, w3_ref, b3_ref, o_ref):
    z = jnp.maximum(y_ref[...] @ w1_ref[...] + b1_ref[...], 0.0)
    z = jnp.maximum(z @ w2_ref[...] + b2_ref[...], 0.0)
    o_ref[...] = jnp.maximum(z @ w3_ref[...] + b3_ref[...], 0.0)


_mlp = pl.pallas_call(
    _mlp_body,
    out_shape=jax.ShapeDtypeStruct((56, H), jnp.float32),
)


def kernel(x, edge_index, batch, W1l, W1r, b1, W2l, W2r, b2, W3l, W3r, b3,
           cw, cb, lw, lb, l2w, l2b, ow, ob):
    f32 = jnp.float32
    h0 = jnp.zeros((NP, H), f32).at[:N].set(jnp.nan_to_num(x))
    srcp = jnp.full((EP,), N, jnp.int32).at[:E].set(edge_index[0])
    dstp = jnp.full((EP,), N, jnp.int32).at[:E].set(edge_index[1])
    batchp = jnp.full((NP, 1), B, jnp.int32).at[:N, 0].set(batch)

    cwm = jnp.zeros((KS * H, H), f32).at[:, :C1].set(
        jnp.transpose(cw, (2, 1, 0)).reshape(KS * H, C1))
    cbp = jnp.zeros((1, H), f32).at[0, :C1].set(cb)
    lw2 = jnp.zeros((C1 * L1, H), f32).at[:, :DL1].set(
        lw.reshape(C1, L1, DL1).transpose(1, 0, 2).reshape(C1 * L1, DL1))
    lbp = jnp.zeros((1, H), f32).at[0, :DL1].set(lb)
    l2wp = jnp.zeros((H, H), f32).at[:DL1, :DL2].set(l2w)
    l2bp = jnp.zeros((1, H), f32).at[0, :DL2].set(l2b)
    owp = jnp.zeros((H, H), f32).at[:DL2, :NUM_OUT].set(ow)
    obp = jnp.zeros((1, H), f32).at[0, :NUM_OUT].set(ob)

    seg = _make_seg_sum()
    layer = _make_layer(False)
    layer_k = _make_layer(True)

    dp3 = _make_deg()(dstp)
    hr = _hr(h0, W1r, b1.reshape(1, H))
    parts = seg(h0, srcp, dstp)
    h1 = layer(parts, hr, dp3, W1l)
    hr = _hr(h1, W2r, b2.reshape(1, H))
    parts = seg(h1, srcp, dstp)
    h2 = layer(parts, hr, dp3, W2l)
    hr = _hr(h2, W3r, b3.reshape(1, H))
    parts = seg(h2, srcp, dstp)
    h3, key = layer_k(parts, hr, dp3, W3l)

    oidx = _sortpool(key, batchp)
    idx_bk = oidx[:K_POOL, :B].T
    win = jnp.stack([idx_bk[:, l:l + KS] for l in range(L1)], axis=1)
    gidx = jnp.concatenate(
        [win.reshape(-1), jnp.full((GR - B * L1 * KS,), N, jnp.int32)])

    rows = _gather_rows()(h3, gidx)
    u = rows[:B * L1 * KS].reshape(B * L1, KS * H)
    up = jnp.zeros((1408, KS * H), f32).at[:B * L1].set(u)
    yconv = _conv(up, cwm, cbp)
    y = yconv[:B * L1, :C1].reshape(B, C1 * L1)
    yp = jnp.zeros((56, C1 * L1), f32).at[:B].set(y)
    out = _mlp(yp, lw2, lbp, l2wp, l2bp, owp, obp)
    return out[:B, :NUM_OUT]


# speedup vs baseline: 1.0708x; 1.0708x over previous
import functools

import jax
import jax.numpy as jnp
from jax import lax
from jax.experimental import pallas as pl
from jax.experimental.pallas import tpu as pltpu
from jax.experimental.pallas import tpu_sc as plsc

N = 10000
E = 320000
H = 128
B = 50
K_POOL = 30
KS = 3
L1 = K_POOL - KS + 1
C1 = 32
DL1 = 32
DL2 = 64
NUM_OUT = 10

NP = 10240
NC, NS = 2, 16
NW = NC * NS
CH = 120
NCH = 84
NPAIR = NCH // 2
EPW = NCH * CH
EP = NW * EPW
NACC = 10112
RTA = NACC // NS
RB = 512
GR = 4608
GPW = GR // NW
_CC = (120, 120, 120, 120, 120, 32)


def _sc_mesh():
    return plsc.VectorSubcoreMesh(
        core_axis_name="c", subcore_axis_name="s", num_cores=NC, num_subcores=NS)


def _fill_tile(buf, w, val):
    v16 = jnp.full((16,), val, jnp.float32)

    def frow(r, _):
        for v in range(w // 16):
            buf[r, pl.ds(v * 16, 16)] = v16
        return _
    lax.fori_loop(0, CH, frow, 0)


def _zero_acc(rows0, w, acc, s):
    _fill_tile(rows0, w, 0.0)
    off = 0
    for sz in _CC:
        pltpu.sync_copy(rows0.at[pl.ds(0, sz)],
                        acc.at[pl.ds(s * RTA + off, sz)])
        off += sz


def _copy_out(acc, out_hbm, c, s, so):
    off = 0
    for sz in _CC:
        r0 = s * RTA + off
        pltpu.async_copy(acc.at[pl.ds(r0, sz)], out_hbm.at[c, pl.ds(r0, sz)], so)
        off += sz
    off = 0
    for sz in _CC:
        pltpu.make_async_copy(acc.at[pl.ds(0, sz)],
                              out_hbm.at[0, pl.ds(0, sz)], so).wait()
        off += sz


GS = (64, 56)


def _make_seg_sum():
    def body(h_hbm, src_hbm, dst_hbm, parts_hbm,
             sb0a, sb0b, sb1a, sb1b, db0a, db0b, db1a, db1b,
             rows0, rows1, acc, si0, si1, sg0, sg1, ss0, ss1, so):
        c = lax.axis_index("c")
        s = lax.axis_index("s")
        wid = s * NC + c
        ebase = wid * EPW
        sbufs = ((sb0a, sb1a), (sb0b, sb1b))
        dbufs = ((db0a, db1a), (db0b, db1b))
        isems = (si0, si1)

        def idx_fetch(p, q):
            for k in range(2):
                base = ebase + (2 * p + k) * CH
                pltpu.async_copy(src_hbm.at[pl.ds(base, CH)], sbufs[q][k], isems[q])
                pltpu.async_copy(dst_hbm.at[pl.ds(base, CH)], dbufs[q][k], isems[q])

        def idx_wait(q):
            for _ in range(4):
                pltpu.make_async_copy(src_hbm.at[pl.ds(0, CH)], sbufs[q][0],
                                      isems[q]).wait()

        def gather(q, k, buf, sem):
            off = 0
            for sz in GS:
                pltpu.async_copy(h_hbm.at[sbufs[q][k].at[pl.ds(off, sz)]],
                                 buf.at[pl.ds(off, sz)], sem)
                off += sz

        def wait_g(buf, sem):
            for sz in GS:
                pltpu.make_async_copy(h_hbm.at[pl.ds(0, sz)],
                                      buf.at[pl.ds(0, sz)], sem).wait()

        def drain_s(buf, sem):
            pltpu.make_async_copy(buf, acc.at[pl.ds(0, CH)], sem).wait()

        idx_fetch(0, 0)
        _zero_acc(rows0, H, acc, s)
        plsc.subcore_barrier()

        def do_pair(p, q):
            idx_wait(q)

            @pl.when(p > 0)
            def _drains():
                drain_s(rows0, ss0)
                drain_s(rows1, ss1)
            gather(q, 0, rows0, sg0)
            gather(q, 1, rows1, sg1)

            @pl.when(p + 1 < NPAIR)
            def _prefetch():
                idx_fetch(p + 1, 1 - q)
            wait_g(rows0, sg0)
            pltpu.async_copy(rows0, acc.at[dbufs[q][0]], ss0, add=True)
            wait_g(rows1, sg1)
            pltpu.async_copy(rows1, acc.at[dbufs[q][1]], ss1, add=True)

        def super_it(u, _):
            do_pair(2 * u, 0)
            do_pair(2 * u + 1, 1)
            return _
        lax.fori_loop(0, NPAIR // 2, super_it, 0)
        drain_s(rows0, ss0)
        drain_s(rows1, ss1)
        plsc.subcore_barrier()

        _copy_out(acc, parts_hbm, c, s, so)

    return pl.kernel(
        body,
        out_type=jax.ShapeDtypeStruct((NC, NP, H), jnp.float32),
        mesh=_sc_mesh(),
        scratch_types=(
            *[pltpu.VMEM((CH,), jnp.int32) for _ in range(8)],
            pltpu.VMEM((CH, H), jnp.float32),
            pltpu.VMEM((CH, H), jnp.float32),
            pltpu.VMEM_SHARED((NACC, H), jnp.float32),
            pltpu.SemaphoreType.DMA,
            pltpu.SemaphoreType.DMA,
            pltpu.SemaphoreType.DMA,
            pltpu.SemaphoreType.DMA,
            pltpu.SemaphoreType.DMA,
            pltpu.SemaphoreType.DMA,
            pltpu.SemaphoreType.DMA,
        ))


def _make_deg():
    def body(dst_hbm, degp_hbm, db0a, db0b, db1a, db1b, ones_tile, dacc,
             si0, si1, ss0, ss1, so):
        c = lax.axis_index("c")
        s = lax.axis_index("s")
        wid = s * NC + c
        ebase = wid * EPW
        dbufs = ((db0a, db1a), (db0b, db1b))
        isems = (si0, si1)

        _zero_acc(ones_tile, H, dacc, s)
        _fill_tile(ones_tile, H, 1.0)
        plsc.subcore_barrier()

        def idx_fetch(p, q):
            for k in range(2):
                base = ebase + (2 * p + k) * CH
                pltpu.async_copy(dst_hbm.at[pl.ds(base, CH)], dbufs[q][k], isems[q])

        def idx_wait(q):
            for _ in range(2):
                pltpu.make_async_copy(dst_hbm.at[pl.ds(0, CH)], dbufs[q][0],
                                      isems[q]).wait()

        def drain_s(sem):
            pltpu.make_async_copy(ones_tile, dacc.at[pl.ds(0, CH)], sem).wait()

        idx_fetch(0, 0)

        def do_pair(p, q):
            idx_wait(q)

            @pl.when(p > 0)
            def _drains():
                drain_s(ss0)
                drain_s(ss1)

            @pl.when(p + 1 < NCH // 2)
            def _prefetch():
                idx_fetch(p + 1, 1 - q)
            pltpu.async_copy(ones_tile, dacc.at[dbufs[q][0]], ss0, add=True)
            pltpu.async_copy(ones_tile, dacc.at[dbufs[q][1]], ss1, add=True)

        def super_it(u, _):
            do_pair(2 * u, 0)
            do_pair(2 * u + 1, 1)
            return _
        lax.fori_loop(0, NCH // 4, super_it, 0)
        drain_s(ss0)
        drain_s(ss1)
        plsc.subcore_barrier()

        _copy_out(dacc, degp_hbm, c, s, so)

    return pl.kernel(
        body,
        out_type=jax.ShapeDtypeStruct((NC, NP, H), jnp.float32),
        mesh=_sc_mesh(),
        scratch_types=(
            pltpu.VMEM((CH,), jnp.int32),
            pltpu.VMEM((CH,), jnp.int32),
            pltpu.VMEM((CH,), jnp.int32),
            pltpu.VMEM((CH,), jnp.int32),
            pltpu.VMEM((CH, H), jnp.float32),
            pltpu.VMEM_SHARED((NACC, H), jnp.float32),
            pltpu.SemaphoreType.DMA,
            pltpu.SemaphoreType.DMA,
            pltpu.SemaphoreType.DMA,
            pltpu.SemaphoreType.DMA,
            pltpu.SemaphoreType.DMA,
        ))


def _gather_rows():
    def body(h_hbm, gidx_hbm, out_hbm, idx_a, idx_b, rows, sem):
        c = lax.axis_index("c")
        s = lax.axis_index("s")
        base = (s * NC + c) * GPW
        half = GPW // 2
        pltpu.sync_copy(gidx_hbm.at[pl.ds(base, half)], idx_a)
        pltpu.sync_copy(gidx_hbm.at[pl.ds(base + half, half)], idx_b)
        pltpu.async_copy(h_hbm.at[idx_a], rows, sem).wait()
        pltpu.sync_copy(rows, out_hbm.at[pl.ds(base, half)])
        pltpu.async_copy(h_hbm.at[idx_b], rows, sem).wait()
        pltpu.sync_copy(rows, out_hbm.at[pl.ds(base + half, half)])

    return pl.kernel(
        body,
        out_type=jax.ShapeDtypeStruct((GR, H), jnp.float32),
        mesh=_sc_mesh(),
        scratch_types=(
            pltpu.VMEM((GPW // 2,), jnp.int32),
            pltpu.VMEM((GPW // 2,), jnp.int32),
            pltpu.VMEM((GPW // 2, H), jnp.float32),
            pltpu.SemaphoreType.DMA,
        ))


def _layer_body(emit_key, hp_ref, h_ref, dp_ref, wl_ref, wr_ref, b_ref, *outs):
    i = pl.program_id(0)
    agg = hp_ref[0] + hp_ref[1]
    d = dp_ref[0, :, 0:1] + dp_ref[1, :, 0:1]
    mean = agg / jnp.maximum(d, 1.0)
    out = mean @ wl_ref[...] + h_ref[...] @ wr_ref[...] + b_ref[...]
    out = jnp.maximum(out, 0.0)
    rid = i * RB + lax.broadcasted_iota(jnp.int32, (RB, 1), 0)
    out = jnp.where(rid < N, out, 0.0)
    outs[0][...] = out
    if emit_key:
        outs[1][...] = out[:, H - 1:H]


def _make_layer(emit_key):
    out_shape = [jax.ShapeDtypeStruct((NP, H), jnp.float32)]
    out_specs = [pl.BlockSpec((RB, H), lambda i: (i, 0))]
    if emit_key:
        out_shape.append(jax.ShapeDtypeStruct((NP, 1), jnp.float32))
        out_specs.append(pl.BlockSpec((RB, 1), lambda i: (i, 0)))
    return pl.pallas_call(
        functools.partial(_layer_body, emit_key),
        grid=(NP // RB,),
        in_specs=[
            pl.BlockSpec((NC, RB, H), lambda i: (0, i, 0)),
            pl.BlockSpec((RB, H), lambda i: (i, 0)),
            pl.BlockSpec((NC, RB, H), lambda i: (0, i, 0)),
            pl.BlockSpec((H, H), lambda i: (0, 0)),
            pl.BlockSpec((H, H), lambda i: (0, 0)),
            pl.BlockSpec((1, H), lambda i: (0, 0)),
        ],
        out_specs=out_specs if emit_key else out_specs[0],
        out_shape=out_shape if emit_key else out_shape[0],
    )


NP2 = NP // 2


def _sortpool_body(key_ref, batch_ref, out_ref, keys_scr):
    jiota = lax.broadcasted_iota(jnp.int32, (NP2, 128), 1)
    riota = lax.broadcasted_iota(jnp.int32, (NP2, 128), 0)
    gcol = jiota & 63
    hi = jiota >= 64
    nodeid = 2 * riota + jnp.where(hi, 1, 0)
    bsel = jnp.where(hi, batch_ref[:, 1:2], batch_ref[:, 0:1])
    ksel = jnp.where(hi, key_ref[:, 1:2], key_ref[:, 0:1])
    keys_scr[...] = jnp.where(bsel == gcol, ksel, -1.0)

    def it(k, out):
        kv = keys_scr[...]
        m = jnp.max(kv, axis=0, keepdims=True)
        mb = jnp.maximum(m[:, :64], m[:, 64:])
        mfull = jnp.concatenate([mb, mb], axis=1)
        cand = jnp.where(kv == mfull, nodeid, NP)
        sel = jnp.min(cand, axis=0, keepdims=True)
        selb = jnp.minimum(sel[:, :64], sel[:, 64:])
        selb = jnp.where(mb > -0.5, selb, N)
        sfull = jnp.concatenate([selb, selb], axis=1)
        keys_scr[...] = jnp.where(nodeid == sfull, -1.0, kv)
        out = jnp.where(
            lax.broadcasted_iota(jnp.int32, (32, 64), 0) == k, selb, out)
        return out

    out_ref[...] = lax.fori_loop(0, K_POOL, it, jnp.zeros((32, 64), jnp.int32))


_sortpool = pl.pallas_call(
    _sortpool_body,
    out_shape=jax.ShapeDtypeStruct((32, 64), jnp.int32),
    scratch_shapes=[pltpu.VMEM((NP2, 128), jnp.float32)],
)


def _conv_body(u_ref, w_ref, b_ref, o_ref):
    o_ref[...] = jnp.maximum(u_ref[...] @ w_ref[...] + b_ref[...], 0.0)


_conv = pl.pallas_call(
    _conv_body,
    out_shape=jax.ShapeDtypeStruct((1408, H), jnp.float32),
)


def _mlp_body(y_ref, w1_ref, b1_ref, w2_ref, b2_ref, w3_ref, b3_ref, o_ref):
    z = jnp.maximum(y_ref[...] @ w1_ref[...] + b1_ref[...], 0.0)
    z = jnp.maximum(z @ w2_ref[...] + b2_ref[...], 0.0)
    o_ref[...] = jnp.maximum(z @ w3_ref[...] + b3_ref[...], 0.0)


_mlp = pl.pallas_call(
    _mlp_body,
    out_shape=jax.ShapeDtypeStruct((56, H), jnp.float32),
)


def kernel(x, edge_index, batch, W1l, W1r, b1, W2l, W2r, b2, W3l, W3r, b3,
           cw, cb, lw, lb, l2w, l2b, ow, ob):
    f32 = jnp.float32
    h0 = jnp.zeros((NP, H), f32).at[:N].set(jnp.nan_to_num(x))
    srcp = jnp.full((EP,), N, jnp.int32).at[:E].set(edge_index[0])
    dstp = jnp.full((EP,), N, jnp.int32).at[:E].set(edge_index[1])
    batchp = jnp.full((NP, 1), B, jnp.int32).at[:N, 0].set(batch)

    cwm = jnp.zeros((KS * H, H), f32).at[:, :C1].set(
        jnp.transpose(cw, (2, 1, 0)).reshape(KS * H, C1))
    cbp = jnp.zeros((1, H), f32).at[0, :C1].set(cb)
    lw2 = jnp.zeros((C1 * L1, H), f32).at[:, :DL1].set(
        lw.reshape(C1, L1, DL1).transpose(1, 0, 2).reshape(C1 * L1, DL1))
    lbp = jnp.zeros((1, H), f32).at[0, :DL1].set(lb)
    l2wp = jnp.zeros((H, H), f32).at[:DL1, :DL2].set(l2w)
    l2bp = jnp.zeros((1, H), f32).at[0, :DL2].set(l2b)
    owp = jnp.zeros((H, H), f32).at[:DL2, :NUM_OUT].set(ow)
    obp = jnp.zeros((1, H), f32).at[0, :NUM_OUT].set(ob)

    seg = _make_seg_sum()
    layer = _make_layer(False)
    layer_k = _make_layer(True)

    dp3 = _make_deg()(dstp)
    parts = seg(h0, srcp, dstp)
    h1 = layer(parts, h0, dp3, W1l, W1r, b1.reshape(1, H))
    parts = seg(h1, srcp, dstp)
    h2 = layer(parts, h1, dp3, W2l, W2r, b2.reshape(1, H))
    parts = seg(h2, srcp, dstp)
    h3, key = layer_k(parts, h2, dp3, W3l, W3r, b3.reshape(1, H))

    oidx = _sortpool(key.reshape(NP2, 2), batchp.reshape(NP2, 2))
    idx_bk = oidx[:K_POOL, :B].T
    win = jnp.stack([idx_bk[:, l:l + KS] for l in range(L1)], axis=1)
    gidx = jnp.concatenate(
        [win.reshape(-1), jnp.full((GR - B * L1 * KS,), N, jnp.int32)])

    rows = _gather_rows()(h3, gidx)
    u = rows[:B * L1 * KS].reshape(B * L1, KS * H)
    up = jnp.zeros((1408, KS * H), f32).at[:B * L1].set(u)
    yconv = _conv(up, cwm, cbp)
    y = yconv[:B * L1, :C1].reshape(B, C1 * L1)
    yp = jnp.zeros((56, C1 * L1), f32).at[:B].set(y)
    out = _mlp(yp, lw2, lbp, l2wp, l2bp, owp, obp)
    return out[:B, :NUM_OUT]
